# baseline (speedup 1.0000x reference)
import functools

import jax
import jax.numpy as jnp
from jax import lax
from jax.experimental import pallas as pl
from jax.experimental.pallas import tpu as pltpu
from jax.experimental.pallas import tpu_sc as plsc

NC = 2
NS = 16
NW = NC * NS
ICHUNK = 2048
LANES = 16


@functools.lru_cache(maxsize=None)
def _build(B, NNEG, D, V):
    rows_per_w = D // NW
    n_ichunks = B // ICHUNK
    v_main = (V // 128) * 128
    v_tail = V - v_main

    mesh = plsc.VectorSubcoreMesh(core_axis_name="c", subcore_axis_name="s")

    def body(dom_i, cod_i, neg0_i, neg1_i, neg2_i, neg3_i, neg4_i,
             in_t, out_t,
             o0, o1, o2,
             row_buf, idx_buf, out_row, didx, tail_blk):
        cid = lax.axis_index("c")
        sid = lax.axis_index("s")
        wid = sid * NC + cid

        def fetch_tails(tab):
            if v_tail:
                pltpu.sync_copy(tab.at[:, pl.ds(v_main, v_tail)], tail_blk)

        def fetch_row(tab, d):
            didx[pl.ds(0, LANES)] = jnp.full((LANES,), d, jnp.int32)
            ivec = didx.at[pl.ds(0, 1)]
            pltpu.sync_copy(tab.at[ivec, pl.ds(0, v_main)],
                            row_buf.at[pl.ds(0, 1), pl.ds(0, v_main)])
            for t0 in range(0, v_tail, LANES):
                tw = min(LANES, v_tail - t0)
                assert tw == LANES, "tail must be a multiple of 16"
                row_buf[0, pl.ds(v_main + t0, LANES)] = (
                    tail_blk[d, pl.ds(t0, LANES)])

        def gather_plane(idx_ref, out_view):
            for ch in range(n_ichunks):
                j0 = ch * ICHUNK
                pltpu.sync_copy(idx_ref.at[pl.ds(j0, ICHUNK)], idx_buf)

                def step(k, _):
                    base = pl.multiple_of(k * LANES, LANES)
                    idxv = idx_buf[pl.ds(base, LANES)]
                    out_row[0, pl.ds(j0 + base, LANES)] = (
                        plsc.load_gather(row_buf.at[0], [idxv]))
                    return 0

                lax.fori_loop(0, ICHUNK // LANES, step, 0, unroll=8)
            pltpu.sync_copy(out_row, out_view.at[didx.at[pl.ds(0, 1)]])

        fetch_tails(in_t)
        for r in range(rows_per_w):
            d = wid * rows_per_w + r
            fetch_row(in_t, d)
            gather_plane(dom_i, o0)
        negs = (neg0_i, neg1_i, neg2_i, neg3_i, neg4_i)
        fetch_tails(out_t)
        for r in range(rows_per_w):
            d = wid * rows_per_w + r
            fetch_row(out_t, d)
            gather_plane(cod_i, o1)
            for n in range(NNEG):
                gather_plane(negs[n], o2.at[n])

    kfn = pl.kernel(
        body,
        out_type=[
            jax.ShapeDtypeStruct((D, B), jnp.float32),
            jax.ShapeDtypeStruct((D, B), jnp.float32),
            jax.ShapeDtypeStruct((NNEG, D, B), jnp.float32),
        ],
        mesh=mesh,
        compiler_params=pltpu.CompilerParams(needs_layout_passes=False),
        scratch_types=[
            pltpu.VMEM((1, V), jnp.float32),
            pltpu.VMEM((ICHUNK,), jnp.int32),
            pltpu.VMEM((1, B), jnp.float32),
            pltpu.VMEM((LANES,), jnp.int32),
            pltpu.VMEM((D, max(V - (V // 128) * 128, 1)), jnp.float32),
        ],
    )
    return kfn


def kernel(domains, codomains, neg_codomains, in_embed, out_embed):
    B = domains.shape[0]
    NNEG = neg_codomains.shape[1]
    V, D = in_embed.shape
    kfn = _build(B, NNEG, D, V)
    in_t = jnp.transpose(in_embed)
    out_t = jnp.transpose(out_embed)
    neg_t = jnp.transpose(neg_codomains)
    negs = [neg_t[n] for n in range(NNEG)]
    o0, o1, o2 = kfn(domains, codomains, *negs, in_t, out_t)
    return (jnp.transpose(o0), jnp.transpose(o1),
            jnp.transpose(o2, (2, 0, 1)))


# speedup vs baseline: 1.4278x; 1.4278x over previous
import functools

import jax
import jax.numpy as jnp
from jax import lax
from jax.experimental import pallas as pl
from jax.experimental.pallas import tpu as pltpu
from jax.experimental.pallas import tpu_sc as plsc

NC = 2
NS = 16
NW = NC * NS
ICHUNK = 2048
HALF = 8192
LANES = 16


@functools.lru_cache(maxsize=None)
def _build(B, NNEG, D, V):
    rows_per_w = D // NW
    n_ichunks = B // ICHUNK
    v_main = (V // 128) * 128
    v_tail = V - v_main

    mesh = plsc.VectorSubcoreMesh(core_axis_name="c", subcore_axis_name="s")

    def body(dom_i, cod_i, neg0_i, neg1_i, neg2_i, neg3_i, neg4_i,
             in_t, out_t,
             o0, o1, o2,
             row_buf, idx0, idx1, ob0, ob1, didx, tail_blk,
             is0, is1, ss0, ss1):
        cid = lax.axis_index("c")
        sid = lax.axis_index("s")
        wid = sid * NC + cid
        ibufs = (idx0, idx1)
        isems = (is0, is1)
        obufs = (ob0, ob1)
        ssems = (ss0, ss1)

        def fetch_tails(tab):
            if v_tail:
                pltpu.sync_copy(tab.at[:, pl.ds(v_main, v_tail)], tail_blk)

        def fetch_row(tab, d):
            didx[pl.ds(0, LANES)] = jnp.full((LANES,), d, jnp.int32)
            ivec = didx.at[pl.ds(0, 1)]
            pltpu.sync_copy(tab.at[ivec, pl.ds(0, v_main)],
                            row_buf.at[pl.ds(0, 1), pl.ds(0, v_main)])
            for t0 in range(0, v_tail, LANES):
                row_buf[0, pl.ds(v_main + t0, LANES)] = (
                    tail_blk[d, pl.ds(t0, LANES)])

        def run_row(tab, d, planes):
            fetch_row(tab, d)
            tasks = [(idx_ref, out_view, c)
                     for idx_ref, out_view in planes
                     for c in range(n_ichunks)]
            nt = len(tasks)
            ih = [None] * nt
            sh = [None, None]

            def start_idx(t):
                idx_ref, _, c = tasks[t]
                ih[t] = pltpu.async_copy(
                    idx_ref.at[pl.ds(c * ICHUNK, ICHUNK)],
                    ibufs[t % 2], isems[t % 2])

            start_idx(0)
            for t in range(nt):
                idx_ref, out_view, c = tasks[t]
                hb = (c * ICHUNK) // HALF
                ob = obufs[hb]
                if t + 1 < nt:
                    start_idx(t + 1)
                ih[t].wait()
                if c * ICHUNK % HALF == 0 and sh[hb] is not None:
                    sh[hb].wait()

                def step(k, _):
                    base = pl.multiple_of(k * LANES, LANES)
                    idxv = ibufs[t % 2][pl.ds(base, LANES)]
                    ob[0, pl.ds(c * ICHUNK % HALF + base, LANES)] = (
                        plsc.load_gather(row_buf.at[0], [idxv]))
                    return 0

                lax.fori_loop(0, ICHUNK // LANES, step, 0, unroll=8)
                if (c + 1) * ICHUNK % HALF == 0:
                    sh[hb] = pltpu.async_copy(
                        ob,
                        out_view.at[didx.at[pl.ds(0, 1)],
                                    pl.ds(hb * HALF, HALF)],
                        ssems[hb])
            for h in sh:
                if h is not None:
                    h.wait()

        for r in range(rows_per_w):
            d = wid * rows_per_w + r
            if r == 0:
                fetch_tails(in_t)
            run_row(in_t, d, [(dom_i, o0)])
        negs = (neg0_i, neg1_i, neg2_i, neg3_i, neg4_i)
        for r in range(rows_per_w):
            d = wid * rows_per_w + r
            if r == 0:
                fetch_tails(out_t)
            run_row(out_t, d, [(cod_i, o1)] +
                    [(negs[n], o2.at[n]) for n in range(NNEG)])

    kfn = pl.kernel(
        body,
        out_type=[
            jax.ShapeDtypeStruct((D, B), jnp.float32),
            jax.ShapeDtypeStruct((D, B), jnp.float32),
            jax.ShapeDtypeStruct((NNEG, D, B), jnp.float32),
        ],
        mesh=mesh,
        compiler_params=pltpu.CompilerParams(needs_layout_passes=False),
        scratch_types=[
            pltpu.VMEM((1, V), jnp.float32),
            pltpu.VMEM((ICHUNK,), jnp.int32),
            pltpu.VMEM((ICHUNK,), jnp.int32),
            pltpu.VMEM((1, HALF), jnp.float32),
            pltpu.VMEM((1, HALF), jnp.float32),
            pltpu.VMEM((LANES,), jnp.int32),
            pltpu.VMEM((D, max(V - (V // 128) * 128, 1)), jnp.float32),
            pltpu.SemaphoreType.DMA,
            pltpu.SemaphoreType.DMA,
            pltpu.SemaphoreType.DMA,
            pltpu.SemaphoreType.DMA,
        ],
    )
    return kfn


def kernel(domains, codomains, neg_codomains, in_embed, out_embed):
    B = domains.shape[0]
    NNEG = neg_codomains.shape[1]
    V, D = in_embed.shape
    kfn = _build(B, NNEG, D, V)
    in_t = jnp.transpose(in_embed)
    out_t = jnp.transpose(out_embed)
    neg_t = jnp.transpose(neg_codomains)
    negs = [neg_t[n] for n in range(NNEG)]
    o0, o1, o2 = kfn(domains, codomains, *negs, in_t, out_t)
    return (jnp.transpose(o0), jnp.transpose(o1),
            jnp.transpose(o2, (2, 0, 1)))


# speedup vs baseline: 2.1825x; 1.5286x over previous
import functools

import jax
import jax.numpy as jnp
from jax import lax
from jax.experimental import pallas as pl
from jax.experimental.pallas import tpu as pltpu
from jax.experimental.pallas import tpu_sc as plsc

NC = 2
NS = 16
NW = NC * NS
ICHUNK = 2048
HALF = 8192
LANES = 16


@functools.lru_cache(maxsize=None)
def _build(B, NNEG, D, V):
    rows_per_w = D // NW
    n_ichunks = B // ICHUNK
    v_main = (V // 128) * 128
    v_tail = V - v_main

    mesh = plsc.VectorSubcoreMesh(core_axis_name="c", subcore_axis_name="s")

    def body(dom_i, cod_i, neg0_i, neg1_i, neg2_i, neg3_i, neg4_i,
             in_t, out_t,
             o0, o1, o2,
             row_buf, idx0, idx1, ob0, ob1, didx, tail_blk,
             is0, is1, ss0, ss1):
        cid = lax.axis_index("c")
        sid = lax.axis_index("s")
        wid = sid * NC + cid
        ibufs = (idx0, idx1)
        isems = (is0, is1)
        obufs = (ob0, ob1)
        ssems = (ss0, ss1)

        def fetch_tails(tab):
            if v_tail:
                pltpu.sync_copy(tab.at[:, pl.ds(v_main, v_tail)], tail_blk)

        def fetch_row(tab, d):
            didx[pl.ds(0, LANES)] = jnp.full((LANES,), d, jnp.int32)
            ivec = didx.at[pl.ds(0, 1)]
            pltpu.sync_copy(tab.at[ivec, pl.ds(0, v_main)],
                            row_buf.at[pl.ds(0, 1), pl.ds(0, v_main)])
            for t0 in range(0, v_tail, LANES):
                row_buf[0, pl.ds(v_main + t0, LANES)] = (
                    tail_blk[d, pl.ds(t0, LANES)])

        def run_row(tab, d, planes):
            fetch_row(tab, d)
            tasks = [(idx_ref, out_view, c)
                     for idx_ref, out_view in planes
                     for c in range(n_ichunks)]
            nt = len(tasks)
            ih = [None] * nt
            sh = [None, None]

            def start_idx(t):
                idx_ref, _, c = tasks[t]
                ih[t] = pltpu.async_copy(
                    idx_ref.at[pl.ds(c * ICHUNK, ICHUNK)],
                    ibufs[t % 2], isems[t % 2])

            start_idx(0)
            for t in range(nt):
                idx_ref, out_view, c = tasks[t]
                hb = (c * ICHUNK) // HALF
                ob = obufs[hb]
                if t + 1 < nt:
                    start_idx(t + 1)
                ih[t].wait()
                if c * ICHUNK % HALF == 0 and sh[hb] is not None:
                    sh[hb].wait()

                @plsc.parallel_loop(0, ICHUNK, LANES, unroll=8)
                def _(base):
                    idxv = ibufs[t % 2][pl.ds(base, LANES)]
                    ob[0, pl.ds(c * ICHUNK % HALF + base, LANES)] = (
                        plsc.load_gather(row_buf.at[0], [idxv]))
                if (c + 1) * ICHUNK % HALF == 0:
                    sh[hb] = pltpu.async_copy(
                        ob,
                        out_view.at[didx.at[pl.ds(0, 1)],
                                    pl.ds(hb * HALF, HALF)],
                        ssems[hb])
            for h in sh:
                if h is not None:
                    h.wait()

        for r in range(rows_per_w):
            d = wid * rows_per_w + r
            if r == 0:
                fetch_tails(in_t)
            run_row(in_t, d, [(dom_i, o0)])
        negs = (neg0_i, neg1_i, neg2_i, neg3_i, neg4_i)
        for r in range(rows_per_w):
            d = wid * rows_per_w + r
            if r == 0:
                fetch_tails(out_t)
            run_row(out_t, d, [(cod_i, o1)] +
                    [(negs[n], o2.at[n]) for n in range(NNEG)])

    kfn = pl.kernel(
        body,
        out_type=[
            jax.ShapeDtypeStruct((D, B), jnp.float32),
            jax.ShapeDtypeStruct((D, B), jnp.float32),
            jax.ShapeDtypeStruct((NNEG, D, B), jnp.float32),
        ],
        mesh=mesh,
        compiler_params=pltpu.CompilerParams(needs_layout_passes=False),
        scratch_types=[
            pltpu.VMEM((1, V), jnp.float32),
            pltpu.VMEM((ICHUNK,), jnp.int32),
            pltpu.VMEM((ICHUNK,), jnp.int32),
            pltpu.VMEM((1, HALF), jnp.float32),
            pltpu.VMEM((1, HALF), jnp.float32),
            pltpu.VMEM((LANES,), jnp.int32),
            pltpu.VMEM((D, max(V - (V // 128) * 128, 1)), jnp.float32),
            pltpu.SemaphoreType.DMA,
            pltpu.SemaphoreType.DMA,
            pltpu.SemaphoreType.DMA,
            pltpu.SemaphoreType.DMA,
        ],
    )
    return kfn


def kernel(domains, codomains, neg_codomains, in_embed, out_embed):
    B = domains.shape[0]
    NNEG = neg_codomains.shape[1]
    V, D = in_embed.shape
    kfn = _build(B, NNEG, D, V)
    in_t = jnp.transpose(in_embed)
    out_t = jnp.transpose(out_embed)
    neg_t = jnp.transpose(neg_codomains)
    negs = [neg_t[n] for n in range(NNEG)]
    o0, o1, o2 = kfn(domains, codomains, *negs, in_t, out_t)
    return (jnp.transpose(o0), jnp.transpose(o1),
            jnp.transpose(o2, (2, 0, 1)))


# speedup vs baseline: 2.4029x; 1.1010x over previous
import functools

import jax
import jax.numpy as jnp
from jax import lax
from jax.experimental import pallas as pl
from jax.experimental.pallas import tpu as pltpu
from jax.experimental.pallas import tpu_sc as plsc

NC = 2
NS = 16
NW = NC * NS
ICHUNK = 2048
HALF = 8192
LANES = 16


@functools.lru_cache(maxsize=None)
def _build(B, NNEG, D, V):
    rows_per_w = D // NW
    n_ichunks = B // ICHUNK
    v_main = (V // 128) * 128
    v_tail = V - v_main

    mesh = plsc.VectorSubcoreMesh(core_axis_name="c", subcore_axis_name="s")

    def body(dom_i, cod_i, neg0_i, neg1_i, neg2_i, neg3_i, neg4_i,
             in_t, out_t,
             o0, o1, o2,
             row_buf, idx0, idx1, idx2, ob0, ob1, didx, tail_blk,
             is0, is1, is2, ss0, ss1):
        cid = lax.axis_index("c")
        sid = lax.axis_index("s")
        wid = sid * NC + cid
        ibufs = (idx0, idx1, idx2)
        isems = (is0, is1, is2)
        obufs = (ob0, ob1)
        ssems = (ss0, ss1)
        NIB = len(ibufs)

        def fetch_tails(tab):
            if v_tail:
                pltpu.sync_copy(tab.at[:, pl.ds(v_main, v_tail)], tail_blk)

        def fetch_row(tab, d):
            didx[pl.ds(0, LANES)] = jnp.full((LANES,), d, jnp.int32)
            ivec = didx.at[pl.ds(0, 1)]
            pltpu.sync_copy(tab.at[ivec, pl.ds(0, v_main)],
                            row_buf.at[pl.ds(0, 1), pl.ds(0, v_main)])
            for t0 in range(0, v_tail, LANES):
                row_buf[0, pl.ds(v_main + t0, LANES)] = (
                    tail_blk[d, pl.ds(t0, LANES)])

        def run_row(tab, d, planes):
            fetch_row(tab, d)
            tasks = [(idx_ref, out_view, c)
                     for idx_ref, out_view in planes
                     for c in range(n_ichunks)]
            nt = len(tasks)
            ih = [None] * nt
            sh = [None, None]

            def start_idx(t):
                idx_ref, _, c = tasks[t]
                ih[t] = pltpu.async_copy(
                    idx_ref.at[pl.ds(c * ICHUNK, ICHUNK)],
                    ibufs[t % NIB], isems[t % NIB])

            for t in range(min(NIB - 1, nt)):
                start_idx(t)
            for t in range(nt):
                idx_ref, out_view, c = tasks[t]
                hb = (c * ICHUNK) // HALF
                ob = obufs[hb]
                if t + NIB - 1 < nt:
                    start_idx(t + NIB - 1)
                ih[t].wait()
                if c * ICHUNK % HALF == 0 and sh[hb] is not None:
                    sh[hb].wait()

                @plsc.parallel_loop(0, ICHUNK, LANES, unroll=8)
                def _(base):
                    idxv = ibufs[t % NIB][pl.ds(base, LANES)]
                    ob[0, pl.ds(c * ICHUNK % HALF + base, LANES)] = (
                        plsc.load_gather(row_buf.at[0], [idxv]))
                if (c + 1) * ICHUNK % HALF == 0:
                    sh[hb] = pltpu.async_copy(
                        ob,
                        out_view.at[didx.at[pl.ds(0, 1)],
                                    pl.ds(hb * HALF, HALF)],
                        ssems[hb])
            for h in sh:
                if h is not None:
                    h.wait()

        for r in range(rows_per_w):
            d = wid * rows_per_w + r
            if r == 0:
                fetch_tails(in_t)
            run_row(in_t, d, [(dom_i, o0)])
        negs = (neg0_i, neg1_i, neg2_i, neg3_i, neg4_i)
        for r in range(rows_per_w):
            d = wid * rows_per_w + r
            if r == 0:
                fetch_tails(out_t)
            run_row(out_t, d, [(cod_i, o1)] +
                    [(negs[n], o2.at[n]) for n in range(NNEG)])

    kfn = pl.kernel(
        body,
        out_type=[
            jax.ShapeDtypeStruct((D, B), jnp.float32),
            jax.ShapeDtypeStruct((D, B), jnp.float32),
            jax.ShapeDtypeStruct((NNEG, D, B), jnp.float32),
        ],
        mesh=mesh,
        compiler_params=pltpu.CompilerParams(needs_layout_passes=False),
        scratch_types=[
            pltpu.VMEM((1, V), jnp.float32),
            pltpu.VMEM((ICHUNK,), jnp.int32),
            pltpu.VMEM((ICHUNK,), jnp.int32),
            pltpu.VMEM((ICHUNK,), jnp.int32),
            pltpu.VMEM((1, HALF), jnp.float32),
            pltpu.VMEM((1, HALF), jnp.float32),
            pltpu.VMEM((LANES,), jnp.int32),
            pltpu.VMEM((D, max(V - (V // 128) * 128, 1)), jnp.float32),
            pltpu.SemaphoreType.DMA,
            pltpu.SemaphoreType.DMA,
            pltpu.SemaphoreType.DMA,
            pltpu.SemaphoreType.DMA,
            pltpu.SemaphoreType.DMA,
        ],
    )
    return kfn


def kernel(domains, codomains, neg_codomains, in_embed, out_embed):
    B = domains.shape[0]
    NNEG = neg_codomains.shape[1]
    V, D = in_embed.shape
    kfn = _build(B, NNEG, D, V)
    in_t = jnp.transpose(in_embed)
    out_t = jnp.transpose(out_embed)
    neg_t = jnp.transpose(neg_codomains)
    negs = [neg_t[n] for n in range(NNEG)]
    o0, o1, o2 = kfn(domains, codomains, *negs, in_t, out_t)
    return (jnp.transpose(o0), jnp.transpose(o1),
            jnp.transpose(o2, (2, 0, 1)))


# speedup vs baseline: 2.4233x; 1.0085x over previous
import functools

import jax
import jax.numpy as jnp
from jax import lax
from jax.experimental import pallas as pl
from jax.experimental.pallas import tpu as pltpu
from jax.experimental.pallas import tpu_sc as plsc

NC = 2
NS = 16
NW = NC * NS
ICHUNK = 2048
HALF = 8192
LANES = 16


@functools.lru_cache(maxsize=None)
def _build(B, NNEG, D, V):
    rows_per_w = D // NW
    n_ichunks = B // ICHUNK
    v_main = (V // 128) * 128
    v_tail = V - v_main

    mesh = plsc.VectorSubcoreMesh(core_axis_name="c", subcore_axis_name="s")

    def body(dom_i, cod_i, neg0_i, neg1_i, neg2_i, neg3_i, neg4_i,
             in_t, out_t,
             o0, o1, o2,
             row_buf, idx0, idx1, idx2, ob0, ob1, didx0, didx1, tail_blk,
             is0, is1, is2, ss0, ss1, rs):
        cid = lax.axis_index("c")
        sid = lax.axis_index("s")
        wid = sid * NC + cid
        ibufs = (idx0, idx1, idx2)
        isems = (is0, is1, is2)
        obufs = (ob0, ob1)
        ssems = (ss0, ss1)
        NIB = len(ibufs)
        didxs = (didx0, didx1)
        sh = [None, None]
        rix = [0]

        def fetch_tails(tab):
            if v_tail:
                pltpu.sync_copy(tab.at[:, pl.ds(v_main, v_tail)], tail_blk)

        def run_row(tab, d, planes):
            didx = didxs[rix[0] % 2]
            rix[0] += 1
            didx[pl.ds(0, LANES)] = jnp.full((LANES,), d, jnp.int32)
            ivec = didx.at[pl.ds(0, 1)]
            rf = pltpu.async_copy(
                tab.at[ivec, pl.ds(0, v_main)],
                row_buf.at[pl.ds(0, 1), pl.ds(0, v_main)], rs)
            tasks = [(idx_ref, out_view, c)
                     for idx_ref, out_view in planes
                     for c in range(n_ichunks)]
            nt = len(tasks)
            ih = [None] * nt

            def start_idx(t):
                idx_ref, _, c = tasks[t]
                ih[t] = pltpu.async_copy(
                    idx_ref.at[pl.ds(c * ICHUNK, ICHUNK)],
                    ibufs[t % NIB], isems[t % NIB])

            for t in range(min(NIB - 1, nt)):
                start_idx(t)
            rf.wait()
            for t0 in range(0, v_tail, LANES):
                row_buf[0, pl.ds(v_main + t0, LANES)] = (
                    tail_blk[d, pl.ds(t0, LANES)])
            for t in range(nt):
                idx_ref, out_view, c = tasks[t]
                hb = (c * ICHUNK) // HALF
                ob = obufs[hb]
                if t + NIB - 1 < nt:
                    start_idx(t + NIB - 1)
                ih[t].wait()
                if c * ICHUNK % HALF == 0 and sh[hb] is not None:
                    sh[hb].wait()

                @plsc.parallel_loop(0, ICHUNK, LANES, unroll=8)
                def _(base):
                    idxv = ibufs[t % NIB][pl.ds(base, LANES)]
                    ob[0, pl.ds(c * ICHUNK % HALF + base, LANES)] = (
                        plsc.load_gather(row_buf.at[0], [idxv]))
                if (c + 1) * ICHUNK % HALF == 0:
                    sh[hb] = pltpu.async_copy(
                        ob,
                        out_view.at[didx.at[pl.ds(0, 1)],
                                    pl.ds(hb * HALF, HALF)],
                        ssems[hb])

        for r in range(rows_per_w):
            d = wid * rows_per_w + r
            if r == 0:
                fetch_tails(in_t)
            run_row(in_t, d, [(dom_i, o0)])
        negs = (neg0_i, neg1_i, neg2_i, neg3_i, neg4_i)
        for r in range(rows_per_w):
            d = wid * rows_per_w + r
            if r == 0:
                fetch_tails(out_t)
            run_row(out_t, d, [(cod_i, o1)] +
                    [(negs[n], o2.at[n]) for n in range(NNEG)])
        for h in sh:
            if h is not None:
                h.wait()

    kfn = pl.kernel(
        body,
        out_type=[
            jax.ShapeDtypeStruct((D, B), jnp.float32),
            jax.ShapeDtypeStruct((D, B), jnp.float32),
            jax.ShapeDtypeStruct((NNEG, D, B), jnp.float32),
        ],
        mesh=mesh,
        compiler_params=pltpu.CompilerParams(needs_layout_passes=False),
        scratch_types=[
            pltpu.VMEM((1, V), jnp.float32),
            pltpu.VMEM((ICHUNK,), jnp.int32),
            pltpu.VMEM((ICHUNK,), jnp.int32),
            pltpu.VMEM((ICHUNK,), jnp.int32),
            pltpu.VMEM((1, HALF), jnp.float32),
            pltpu.VMEM((1, HALF), jnp.float32),
            pltpu.VMEM((LANES,), jnp.int32),
            pltpu.VMEM((LANES,), jnp.int32),
            pltpu.VMEM((D, max(V - (V // 128) * 128, 1)), jnp.float32),
            pltpu.SemaphoreType.DMA,
            pltpu.SemaphoreType.DMA,
            pltpu.SemaphoreType.DMA,
            pltpu.SemaphoreType.DMA,
            pltpu.SemaphoreType.DMA,
            pltpu.SemaphoreType.DMA,
        ],
    )
    return kfn


def kernel(domains, codomains, neg_codomains, in_embed, out_embed):
    B = domains.shape[0]
    NNEG = neg_codomains.shape[1]
    V, D = in_embed.shape
    kfn = _build(B, NNEG, D, V)
    in_t = jnp.transpose(in_embed)
    out_t = jnp.transpose(out_embed)
    neg_t = jnp.transpose(neg_codomains)
    negs = [neg_t[n] for n in range(NNEG)]
    o0, o1, o2 = kfn(domains, codomains, *negs, in_t, out_t)
    return (jnp.transpose(o0), jnp.transpose(o1),
            jnp.transpose(o2, (2, 0, 1)))
